# Initial kernel scaffold; baseline (speedup 1.0000x reference)
import jax
import jax.numpy as jnp
from jax.experimental import pallas as pl


def kernel(lattices, inv_lattices, forces, batch, num_atoms, general_ops, symm_map, num_general_ops):
    raise NotImplementedError("write your pallas kernel here")


# speedup vs baseline: 64.0715x; 64.0715x over previous
import jax
import jax.numpy as jnp
from jax import lax
from jax.experimental import pallas as pl
from jax.experimental.pallas import tpu as pltpu
from jax.experimental.pallas import tpu_sc as plsc

NC = 2
NS = 16
NW = NC * NS
L = 16


def _sc_body(fx, fy, fz, smap, opsr, invr, latr, nopr,
             ox, oy, oz,
             fxv, fyv, fzv, smv, opsv, invv, latv, nopv,
             accx, accy, accz):
    NA = fxv.shape[0] // L
    NOP = smv.shape[0] // (NA * L)

    wid = lax.axis_index("c") * NS + lax.axis_index("s")
    fbase = wid * NA * L
    sbase = wid * NOP * NA * L
    obase = wid * NOP * 16 * L
    lbase = wid * 9 * L
    nbase = wid * L

    pltpu.sync_copy(fx.at[pl.ds(fbase, NA * L)], fxv)
    pltpu.sync_copy(fy.at[pl.ds(fbase, NA * L)], fyv)
    pltpu.sync_copy(fz.at[pl.ds(fbase, NA * L)], fzv)
    pltpu.sync_copy(smap.at[pl.ds(sbase, NOP * NA * L)], smv)
    pltpu.sync_copy(opsr.at[pl.ds(obase, NOP * 16 * L)], opsv)
    pltpu.sync_copy(invr.at[pl.ds(lbase, 9 * L)], invv)
    pltpu.sync_copy(latr.at[pl.ds(lbase, 9 * L)], latv)
    pltpu.sync_copy(nopr.at[pl.ds(nbase, L)], nopv)

    zero = jnp.zeros((L,), jnp.float32)

    @pl.loop(0, NA * L, step=L, unroll=8)
    def _zero(i):
        accx[pl.ds(i, L)] = zero
        accy[pl.ds(i, L)] = zero
        accz[pl.ds(i, L)] = zero

    lane256 = lax.iota(jnp.int32, L) * NA

    scale = 1.0 / nopv[pl.ds(0, L)].astype(jnp.float32)
    inv_s = [[invv[pl.ds((j * 3 + l) * L, L)] * scale for l in range(3)]
             for j in range(3)]
    lat_v = [[latv[pl.ds((k * 3 + i) * L, L)] for i in range(3)]
             for k in range(3)]

    for o in range(NOP):
        r_v = [[opsv[pl.ds((o * 16 + k * 4 + l) * L, L)] for l in range(3)]
               for k in range(3)]
        m = [[None] * 3 for _ in range(3)]
        for j in range(3):
            t1 = [r_v[k][0] * inv_s[j][0] + r_v[k][1] * inv_s[j][1]
                  + r_v[k][2] * inv_s[j][2] for k in range(3)]
            for i in range(3):
                m[j][i] = (t1[0] * lat_v[0][i] + t1[1] * lat_v[1][i]
                           + t1[2] * lat_v[2][i])

        smap_o = o * NA * L

        @pl.loop(0, NA, unroll=4)
        def _atoms(a):
            al = a * L
            f0 = fxv[pl.ds(al, L)]
            f1 = fyv[pl.ds(al, L)]
            f2 = fzv[pl.ds(al, L)]
            idx = smv[pl.ds(smap_o + al, L)] + lane256
            gx = f0 * m[0][0] + f1 * m[1][0] + f2 * m[2][0]
            gy = f0 * m[0][1] + f1 * m[1][1] + f2 * m[2][1]
            gz = f0 * m[0][2] + f1 * m[1][2] + f2 * m[2][2]
            plsc.addupdate_scatter(accx, [idx], gx)
            plsc.addupdate_scatter(accy, [idx], gy)
            plsc.addupdate_scatter(accz, [idx], gz)

    pltpu.sync_copy(accx, ox.at[pl.ds(fbase, NA * L)])
    pltpu.sync_copy(accy, oy.at[pl.ds(fbase, NA * L)])
    pltpu.sync_copy(accz, oz.at[pl.ds(fbase, NA * L)])


def kernel(lattices, inv_lattices, forces, batch, num_atoms, general_ops,
           symm_map, num_general_ops):
    B = lattices.shape[0]
    NOP = symm_map.shape[1]
    NA = symm_map.shape[2]
    N = forces.shape[0]
    SPW = B // NW

    f_t = (forces.reshape(NW, SPW, NA, 3).transpose(3, 0, 2, 1)
           .reshape(3, N))
    smap_t = (symm_map.reshape(NW, SPW, NOP, NA).transpose(0, 2, 3, 1)
              .reshape(-1))
    ops_t = (general_ops.reshape(NW, SPW, NOP, 16).transpose(0, 2, 3, 1)
             .reshape(-1))
    inv_t = (inv_lattices.reshape(NW, SPW, 9).transpose(0, 2, 1)
             .reshape(-1))
    lat_t = (lattices.reshape(NW, SPW, 9).transpose(0, 2, 1)
             .reshape(-1))
    nop_t = num_general_ops.reshape(NW, SPW).reshape(-1)

    mesh = plsc.VectorSubcoreMesh(core_axis_name="c", subcore_axis_name="s",
                                  num_cores=NC, num_subcores=NS)
    out = jax.ShapeDtypeStruct((N,), jnp.float32)
    run = pl.kernel(
        _sc_body,
        out_type=(out, out, out),
        mesh=mesh,
        compiler_params=pltpu.CompilerParams(needs_layout_passes=False),
        scratch_types=[
            pltpu.VMEM((NA * SPW,), jnp.float32),
            pltpu.VMEM((NA * SPW,), jnp.float32),
            pltpu.VMEM((NA * SPW,), jnp.float32),
            pltpu.VMEM((NOP * NA * SPW,), jnp.int32),
            pltpu.VMEM((NOP * 16 * SPW,), jnp.float32),
            pltpu.VMEM((9 * SPW,), jnp.float32),
            pltpu.VMEM((9 * SPW,), jnp.float32),
            pltpu.VMEM((SPW,), jnp.int32),
            pltpu.VMEM((NA * SPW,), jnp.float32),
            pltpu.VMEM((NA * SPW,), jnp.float32),
            pltpu.VMEM((NA * SPW,), jnp.float32),
        ],
    )
    ox, oy, oz = run(f_t[0], f_t[1], f_t[2], smap_t, ops_t, inv_t, lat_t,
                     nop_t)
    return jnp.stack([ox, oy, oz], axis=-1)
